# baseline (speedup 1.0000x reference)
import functools
import math

import jax
import jax.numpy as jnp
from jax import lax
from jax.experimental import pallas as pl
from jax.experimental.pallas import tpu as pltpu
from jax.experimental.pallas import tpu_sc as plsc

N_HEADS = 32
N_KV = 8
HD = 128
HALF = HD // 2
T = 8192
NUM_SLOTS = 32768
NEG_LOG_BASE_OVER_HALF = -math.log(10000.0) / HALF

TOK_CHUNK = 256
N_TOK_CHUNKS = T // TOK_CHUNK

NW = 32
SPW = NUM_SLOTS // NW
NVEC = T // 16
LIST_LEN = SPW + 32
CPR = 32

TC_COPY_START = NUM_SLOTS // 2
CPW = TC_COPY_START // NW
NCH = CPW // CPR
TC_COPY_BLK = 512
N_TC_COPY_BLKS = (NUM_SLOTS - TC_COPY_START) // TC_COPY_BLK


def _rope_body(pos_ref, q_ref, k_ref, v_ref, qo_ref, ko_ref, vo_ref):
    pos = pos_ref[:]
    j = lax.broadcasted_iota(jnp.int32, (TOK_CHUNK, HALF), 1).astype(jnp.float32)
    inv_freq = jnp.exp(j * NEG_LOG_BASE_OVER_HALF)
    freqs = pos * inv_freq
    c = jnp.cos(freqs)[:, None, :]
    s = jnp.sin(freqs)[:, None, :]

    q = q_ref[:].reshape(TOK_CHUNK, N_HEADS, HD)
    q1 = q[:, :, :HALF]
    q2 = q[:, :, HALF:]
    qo_ref[:] = jnp.concatenate([q1 * c - q2 * s, q2 * c + q1 * s], axis=-1)

    k = k_ref[:].reshape(TOK_CHUNK, N_KV, HD)
    k1 = k[:, :, :HALF]
    k2 = k[:, :, HALF:]
    ko_ref[:] = jnp.concatenate([k1 * c - k2 * s, k2 * c + k1 * s], axis=-1)

    vo_ref[:] = v_ref[:].reshape(TOK_CHUNK, N_KV, HD)


_rope_call = pl.pallas_call(
    _rope_body,
    grid=(N_TOK_CHUNKS,),
    in_specs=[
        pl.BlockSpec((TOK_CHUNK, 1), lambda g: (g, 0)),
        pl.BlockSpec((TOK_CHUNK, N_HEADS * HD), lambda g: (g, 0)),
        pl.BlockSpec((TOK_CHUNK, N_KV * HD), lambda g: (g, 0)),
        pl.BlockSpec((TOK_CHUNK, N_KV * HD), lambda g: (g, 0)),
    ],
    out_specs=[
        pl.BlockSpec((TOK_CHUNK, N_HEADS, HD), lambda g: (g, 0, 0)),
        pl.BlockSpec((TOK_CHUNK, N_KV, HD), lambda g: (g, 0, 0)),
        pl.BlockSpec((TOK_CHUNK, N_KV, HD), lambda g: (g, 0, 0)),
    ],
    out_shape=[
        jax.ShapeDtypeStruct((T, N_HEADS, HD), jnp.float32),
        jax.ShapeDtypeStruct((T, N_KV, HD), jnp.float32),
        jax.ShapeDtypeStruct((T, N_KV, HD), jnp.float32),
    ],
    compiler_params=pltpu.CompilerParams(
        dimension_semantics=("arbitrary",),
    ),
)


_TC_COPY_CHUNKS = 8
_TC_COPY_ROWS = (NUM_SLOTS - TC_COPY_START) // _TC_COPY_CHUNKS


def _tc_copy_body(kc_ref, vc_ref, kco_ref, vco_ref, sem):
    descs = []
    for src, dst in ((kc_ref, kco_ref), (vc_ref, vco_ref)):
        for c in range(_TC_COPY_CHUNKS):
            sl = pl.ds(TC_COPY_START + c * _TC_COPY_ROWS, _TC_COPY_ROWS)
            descs.append(pltpu.async_copy(src.at[sl], dst.at[sl], sem))
    for d in descs:
        d.wait()


_tc_copy_call = pl.pallas_call(
    _tc_copy_body,
    in_specs=[
        pl.BlockSpec(memory_space=pltpu.MemorySpace.HBM),
        pl.BlockSpec(memory_space=pltpu.MemorySpace.HBM),
    ],
    out_specs=[
        pl.BlockSpec(memory_space=pltpu.MemorySpace.HBM),
        pl.BlockSpec(memory_space=pltpu.MemorySpace.HBM),
    ],
    out_shape=[
        jax.ShapeDtypeStruct((NUM_SLOTS, N_KV, HD), jnp.float32),
        jax.ShapeDtypeStruct((NUM_SLOTS, N_KV, HD), jnp.float32),
    ],
    scratch_shapes=[pltpu.SemaphoreType.DMA],
)


def _worker_id():
    return lax.axis_index("c") * 16 + lax.axis_index("s")


def _copy_body(kc, vc, slots, kco, vco, wslot_h, wtok_h, cnt_h,
               slots_v, inv_v, wslot_v, wtok_v, cnt_v,
               cbuf0, cbuf1, sem_in, sem_out0, sem_out1, sem_slots):
    wid = _worker_id()
    base = wid * SPW
    cbase = wid * CPW

    sd = pltpu.async_copy(slots, slots_v, sem_slots)

    def copy_range(src, dst):
        bufs = (cbuf0, cbuf1)
        sems = (sem_out0, sem_out1)
        d_out = [None, None]
        for c in range(NCH):
            b = c % 2
            if d_out[b] is not None:
                d_out[b].wait()
            sl = pl.ds(cbase + c * CPR, CPR)
            pltpu.async_copy(src.at[sl], bufs[b], sem_in).wait()
            d_out[b] = pltpu.async_copy(bufs[b], dst.at[sl], sems[b])
        d_out[0].wait()
        d_out[1].wait()

    copy_range(kc, kco)
    copy_range(vc, vco)

    sd.wait()

    iota16 = lax.broadcasted_iota(jnp.int32, (16,), 0)
    neg1 = jnp.full((16,), -1, jnp.int32)

    def init_body(i, carry):
        inv_v[pl.ds(i * 16, 16)] = neg1
        return carry

    lax.fori_loop(0, SPW // 16, init_body, 0)

    def scan_body(i, carry):
        sv = slots_v[pl.ds(i * 16, 16)]
        m = (sv >= base) & (sv < base + SPW)
        _, lastm = plsc.scan_count(sv, mask=m)
        idx = jnp.where(m, sv - base, 0)
        plsc.store_scatter(inv_v, [idx], i * 16 + iota16, mask=lastm & m)
        return carry

    lax.fori_loop(0, NVEC, scan_body, 0)

    def comp_body(j, off):
        iv = inv_v[pl.ds(j * 16, 16)]
        m = iv >= 0
        plsc.store_compressed(wslot_v.at[pl.ds(off, 16)],
                              base + j * 16 + iota16, mask=m)
        plsc.store_compressed(wtok_v.at[pl.ds(off, 16)], iv, mask=m)
        return off + jnp.sum(m.astype(jnp.int32))

    n_w = lax.fori_loop(0, SPW // 16, comp_body, 0)

    n16 = jnp.where(n_w > 0, ((n_w - 1) // 16) * 16 + 16, 0)
    n32 = jnp.where(lax.rem(n16, 32) == 16, n16 + 16, n16)

    @pl.when(n_w > 0)
    def _():
        wbase = n16 - 16
        lastlane = (n_w - 1) - wbase
        ws = wslot_v[pl.ds(wbase, 16)]
        wt = wtok_v[pl.ds(wbase, 16)]
        sel = iota16 == lastlane
        s_safe = jnp.sum(jnp.where(sel, ws, 0))
        t_safe = jnp.sum(jnp.where(sel, wt, 0))
        keep = iota16 <= lastlane
        ones = jnp.full((16,), 1, jnp.int32)
        wslot_v[pl.ds(wbase, 16)] = jnp.where(keep, ws, ones * s_safe)
        wtok_v[pl.ds(wbase, 16)] = jnp.where(keep, wt, ones * t_safe)

        @pl.when(n32 > n16)
        def _():
            wslot_v[pl.ds(n16, 16)] = ones * s_safe
            wtok_v[pl.ds(n16, 16)] = ones * t_safe

        pltpu.sync_copy(wslot_v, wslot_h.at[wid])
        pltpu.sync_copy(wtok_v, wtok_h.at[wid])

    cnt_v[...] = jnp.full((16,), 1, jnp.int32) * n32
    pltpu.sync_copy(cnt_v, cnt_h.at[wid])


_copy_call = functools.partial(
    pl.kernel,
    out_type=(
        jax.ShapeDtypeStruct((NW, LIST_LEN), jnp.int32),
        jax.ShapeDtypeStruct((NW, LIST_LEN), jnp.int32),
        jax.ShapeDtypeStruct((NW, 16), jnp.int32),
    ),
    mesh=plsc.VectorSubcoreMesh(core_axis_name="c", subcore_axis_name="s"),
    compiler_params=pltpu.CompilerParams(
        needs_layout_passes=False,
        use_tc_tiling_on_sc=True,
    ),
    scratch_types=[
        pltpu.VMEM((T,), jnp.int32),
        pltpu.VMEM((SPW,), jnp.int32),
        pltpu.VMEM((LIST_LEN,), jnp.int32),
        pltpu.VMEM((LIST_LEN,), jnp.int32),
        pltpu.VMEM((16,), jnp.int32),
        pltpu.VMEM((CPR, N_KV, HD), jnp.float32),
        pltpu.VMEM((CPR, N_KV, HD), jnp.float32),
        pltpu.SemaphoreType.DMA,
        pltpu.SemaphoreType.DMA,
        pltpu.SemaphoreType.DMA,
        pltpu.SemaphoreType.DMA,
    ],
)(_copy_body)


def _scatter_body(krot, v, wslot_h, wtok_h, cnt_h, kco, vco,
                  wslot_v, wtok_v, cnt_v,
                  krowsA, vrowsA, krowsB, vrowsB,
                  gsemA, gsemB, ssemA, ssemB):
    wid = _worker_id()

    pltpu.sync_copy(wslot_h.at[wid], wslot_v)
    pltpu.sync_copy(wtok_h.at[wid], wtok_v)
    pltpu.sync_copy(cnt_h.at[wid], cnt_v)
    n32 = jnp.max(cnt_v[...])
    trips = n32 // 32

    bufs = ((krowsA, vrowsA, gsemA, ssemA), (krowsB, vrowsB, gsemB, ssemB))

    def trip_body(g, carry):
        for b, (krows, vrows, gsem, ssem) in enumerate(bufs):
            c = 2 * g + b
            tv = wtok_v[pl.ds(c * 16, 16)]
            sv2 = wslot_v[pl.ds(c * 16, 16)]

            @pl.when(g > 0)
            def _():
                pltpu.make_async_copy(krows, kco.at[sv2], ssem).wait()
                pltpu.make_async_copy(vrows, vco.at[sv2], ssem).wait()

            gk = pltpu.async_copy(krot.at[tv], krows, gsem)
            gv = pltpu.async_copy(v.at[tv], vrows, gsem)
            gk.wait()
            gv.wait()
            pltpu.async_copy(krows, kco.at[sv2], ssem)
            pltpu.async_copy(vrows, vco.at[sv2], ssem)
        return carry

    lax.fori_loop(0, trips, trip_body, 0)

    @pl.when(trips > 0)
    def _():
        dummy = wslot_v[pl.ds(0, 16)]
        for krows, vrows, _, ssem in bufs:
            pltpu.make_async_copy(krows, kco.at[dummy], ssem).wait()
            pltpu.make_async_copy(vrows, vco.at[dummy], ssem).wait()


_scatter_call = functools.partial(
    pl.kernel,
    out_type=(),
    mesh=plsc.VectorSubcoreMesh(core_axis_name="c", subcore_axis_name="s"),
    compiler_params=pltpu.CompilerParams(
        needs_layout_passes=False,
        use_tc_tiling_on_sc=True,
    ),
    scratch_types=[
        pltpu.VMEM((LIST_LEN,), jnp.int32),
        pltpu.VMEM((LIST_LEN,), jnp.int32),
        pltpu.VMEM((16,), jnp.int32),
        pltpu.VMEM((16, N_KV, HD), jnp.float32),
        pltpu.VMEM((16, N_KV, HD), jnp.float32),
        pltpu.VMEM((16, N_KV, HD), jnp.float32),
        pltpu.VMEM((16, N_KV, HD), jnp.float32),
        pltpu.SemaphoreType.DMA,
        pltpu.SemaphoreType.DMA,
        pltpu.SemaphoreType.DMA,
        pltpu.SemaphoreType.DMA,
    ],
)(_scatter_body)


def kernel(q, k, v, positions, key_cache, value_cache, slot_mapping):
    posf = positions.astype(jnp.float32).reshape(T, 1)
    sm = slot_mapping.astype(jnp.int32)

    kco, vco = _tc_copy_call(key_cache, value_cache)
    kc_ref = jax.new_ref(kco)
    vc_ref = jax.new_ref(vco)
    wslot_h, wtok_h, cnt_h = _copy_call(key_cache, value_cache, sm,
                                        kc_ref, vc_ref)
    q_out, k_out, v_out = _rope_call(posf, q, k, v)

    _scatter_call(k_out, v_out, wslot_h, wtok_h, cnt_h, kc_ref, vc_ref)
    return (q_out, k_out, v_out, kc_ref[...], vc_ref[...])


# speedup vs baseline: 11.2210x; 11.2210x over previous
import functools
import math

import jax
import jax.numpy as jnp
from jax import lax
from jax.experimental import pallas as pl
from jax.experimental.pallas import tpu as pltpu
from jax.experimental.pallas import tpu_sc as plsc

N_HEADS = 32
N_KV = 8
HD = 128
HALF = HD // 2
T = 8192
NUM_SLOTS = 32768
NEG_LOG_BASE_OVER_HALF = -math.log(10000.0) / HALF

TOK_CHUNK = 256
N_TOK_CHUNKS = T // TOK_CHUNK

NW = 32
SPW = NUM_SLOTS // NW
NVEC = T // 16
LIST_LEN = SPW + 32
CPR = 32

TC_COPY_START = NUM_SLOTS // 2
CPW = TC_COPY_START // NW
NCH = CPW // CPR
TC_COPY_BLK = 1024
N_TC_COPY_BLKS = (NUM_SLOTS - TC_COPY_START) // TC_COPY_BLK


def _rope_body(pos_ref, q_ref, k_ref, v_ref, qo_ref, ko_ref, vo_ref):
    pos = pos_ref[:]
    j = lax.broadcasted_iota(jnp.int32, (TOK_CHUNK, HALF), 1).astype(jnp.float32)
    inv_freq = jnp.exp(j * NEG_LOG_BASE_OVER_HALF)
    freqs = pos * inv_freq
    c = jnp.cos(freqs)[:, None, :]
    s = jnp.sin(freqs)[:, None, :]

    q = q_ref[:].reshape(TOK_CHUNK, N_HEADS, HD)
    q1 = q[:, :, :HALF]
    q2 = q[:, :, HALF:]
    qo_ref[:] = jnp.concatenate([q1 * c - q2 * s, q2 * c + q1 * s], axis=-1)

    k = k_ref[:].reshape(TOK_CHUNK, N_KV, HD)
    k1 = k[:, :, :HALF]
    k2 = k[:, :, HALF:]
    ko_ref[:] = jnp.concatenate([k1 * c - k2 * s, k2 * c + k1 * s], axis=-1)

    vo_ref[:] = v_ref[:].reshape(TOK_CHUNK, N_KV, HD)


_rope_call = pl.pallas_call(
    _rope_body,
    grid=(N_TOK_CHUNKS,),
    in_specs=[
        pl.BlockSpec((TOK_CHUNK, 1), lambda g: (g, 0)),
        pl.BlockSpec((TOK_CHUNK, N_HEADS * HD), lambda g: (g, 0)),
        pl.BlockSpec((TOK_CHUNK, N_KV * HD), lambda g: (g, 0)),
        pl.BlockSpec((TOK_CHUNK, N_KV * HD), lambda g: (g, 0)),
    ],
    out_specs=[
        pl.BlockSpec((TOK_CHUNK, N_HEADS, HD), lambda g: (g, 0, 0)),
        pl.BlockSpec((TOK_CHUNK, N_KV, HD), lambda g: (g, 0, 0)),
        pl.BlockSpec((TOK_CHUNK, N_KV, HD), lambda g: (g, 0, 0)),
    ],
    out_shape=[
        jax.ShapeDtypeStruct((T, N_HEADS, HD), jnp.float32),
        jax.ShapeDtypeStruct((T, N_KV, HD), jnp.float32),
        jax.ShapeDtypeStruct((T, N_KV, HD), jnp.float32),
    ],
    compiler_params=pltpu.CompilerParams(
        dimension_semantics=("arbitrary",),
    ),
)


def _tc_copy_body(kc_ref, vc_ref, kco_ref, vco_ref):
    kco_ref[:] = kc_ref[:]
    vco_ref[:] = vc_ref[:]


_tc_upper_idx = lambda g: (g + TC_COPY_START // TC_COPY_BLK, 0, 0)

_tc_copy_call = pl.pallas_call(
    _tc_copy_body,
    grid=(N_TC_COPY_BLKS,),
    in_specs=[
        pl.BlockSpec((TC_COPY_BLK, N_KV, HD), _tc_upper_idx),
        pl.BlockSpec((TC_COPY_BLK, N_KV, HD), _tc_upper_idx),
    ],
    out_specs=[
        pl.BlockSpec((TC_COPY_BLK, N_KV, HD), _tc_upper_idx),
        pl.BlockSpec((TC_COPY_BLK, N_KV, HD), _tc_upper_idx),
    ],
    out_shape=[
        jax.ShapeDtypeStruct((NUM_SLOTS, N_KV, HD), jnp.float32),
        jax.ShapeDtypeStruct((NUM_SLOTS, N_KV, HD), jnp.float32),
    ],
    compiler_params=pltpu.CompilerParams(
        dimension_semantics=("arbitrary",),
    ),
)


def _worker_id():
    return lax.axis_index("c") * 16 + lax.axis_index("s")


def _copy_body(kc, vc, slots, kco, vco, wslot_h, wtok_h, cnt_h,
               slots_v, inv_v, wslot_v, wtok_v, cnt_v,
               cbuf0, cbuf1, sem_in, sem_out0, sem_out1, sem_slots):
    wid = _worker_id()
    base = wid * SPW
    cbase = wid * CPW

    sd = pltpu.async_copy(slots, slots_v, sem_slots)

    def copy_range(src, dst):
        bufs = (cbuf0, cbuf1)
        sems = (sem_out0, sem_out1)
        d_out = [None, None]
        for c in range(NCH):
            b = c % 2
            if d_out[b] is not None:
                d_out[b].wait()
            sl = pl.ds(cbase + c * CPR, CPR)
            pltpu.async_copy(src.at[sl], bufs[b], sem_in).wait()
            d_out[b] = pltpu.async_copy(bufs[b], dst.at[sl], sems[b])
        d_out[0].wait()
        d_out[1].wait()

    copy_range(kc, kco)
    copy_range(vc, vco)

    sd.wait()

    iota16 = lax.broadcasted_iota(jnp.int32, (16,), 0)
    neg1 = jnp.full((16,), -1, jnp.int32)

    def init_body(i, carry):
        inv_v[pl.ds(i * 16, 16)] = neg1
        return carry

    lax.fori_loop(0, SPW // 16, init_body, 0)

    def scan_body(i, carry):
        sv = slots_v[pl.ds(i * 16, 16)]
        m = (sv >= base) & (sv < base + SPW)
        _, lastm = plsc.scan_count(sv, mask=m)
        idx = jnp.where(m, sv - base, 0)
        plsc.store_scatter(inv_v, [idx], i * 16 + iota16, mask=lastm & m)
        return carry

    lax.fori_loop(0, NVEC, scan_body, 0)

    def comp_body(j, off):
        iv = inv_v[pl.ds(j * 16, 16)]
        m = iv >= 0
        plsc.store_compressed(wslot_v.at[pl.ds(off, 16)],
                              base + j * 16 + iota16, mask=m)
        plsc.store_compressed(wtok_v.at[pl.ds(off, 16)], iv, mask=m)
        return off + jnp.sum(m.astype(jnp.int32))

    n_w = lax.fori_loop(0, SPW // 16, comp_body, 0)

    n16 = jnp.where(n_w > 0, ((n_w - 1) // 16) * 16 + 16, 0)
    n32 = jnp.where(lax.rem(n16, 32) == 16, n16 + 16, n16)

    @pl.when(n_w > 0)
    def _():
        wbase = n16 - 16
        lastlane = (n_w - 1) - wbase
        ws = wslot_v[pl.ds(wbase, 16)]
        wt = wtok_v[pl.ds(wbase, 16)]
        sel = iota16 == lastlane
        s_safe = jnp.sum(jnp.where(sel, ws, 0))
        t_safe = jnp.sum(jnp.where(sel, wt, 0))
        keep = iota16 <= lastlane
        ones = jnp.full((16,), 1, jnp.int32)
        wslot_v[pl.ds(wbase, 16)] = jnp.where(keep, ws, ones * s_safe)
        wtok_v[pl.ds(wbase, 16)] = jnp.where(keep, wt, ones * t_safe)

        @pl.when(n32 > n16)
        def _():
            wslot_v[pl.ds(n16, 16)] = ones * s_safe
            wtok_v[pl.ds(n16, 16)] = ones * t_safe

        pltpu.sync_copy(wslot_v, wslot_h.at[wid])
        pltpu.sync_copy(wtok_v, wtok_h.at[wid])

    cnt_v[...] = jnp.full((16,), 1, jnp.int32) * n32
    pltpu.sync_copy(cnt_v, cnt_h.at[wid])


_copy_call = functools.partial(
    pl.kernel,
    out_type=(
        jax.ShapeDtypeStruct((NW, LIST_LEN), jnp.int32),
        jax.ShapeDtypeStruct((NW, LIST_LEN), jnp.int32),
        jax.ShapeDtypeStruct((NW, 16), jnp.int32),
    ),
    mesh=plsc.VectorSubcoreMesh(core_axis_name="c", subcore_axis_name="s"),
    compiler_params=pltpu.CompilerParams(
        needs_layout_passes=False,
        use_tc_tiling_on_sc=True,
    ),
    scratch_types=[
        pltpu.VMEM((T,), jnp.int32),
        pltpu.VMEM((SPW,), jnp.int32),
        pltpu.VMEM((LIST_LEN,), jnp.int32),
        pltpu.VMEM((LIST_LEN,), jnp.int32),
        pltpu.VMEM((16,), jnp.int32),
        pltpu.VMEM((CPR, N_KV, HD), jnp.float32),
        pltpu.VMEM((CPR, N_KV, HD), jnp.float32),
        pltpu.SemaphoreType.DMA,
        pltpu.SemaphoreType.DMA,
        pltpu.SemaphoreType.DMA,
        pltpu.SemaphoreType.DMA,
    ],
)(_copy_body)


def _scatter_body(krot, v, wslot_h, wtok_h, cnt_h, kco, vco,
                  wslot_v, wtok_v, cnt_v,
                  krowsA, vrowsA, krowsB, vrowsB,
                  gsemA, gsemB, ssemA, ssemB):
    wid = _worker_id()

    pltpu.sync_copy(wslot_h.at[wid], wslot_v)
    pltpu.sync_copy(wtok_h.at[wid], wtok_v)
    pltpu.sync_copy(cnt_h.at[wid], cnt_v)
    n32 = jnp.max(cnt_v[...])
    trips = n32 // 32

    bufs = ((krowsA, vrowsA, gsemA, ssemA), (krowsB, vrowsB, gsemB, ssemB))

    def trip_body(g, carry):
        for b, (krows, vrows, gsem, ssem) in enumerate(bufs):
            c = 2 * g + b
            tv = wtok_v[pl.ds(c * 16, 16)]
            sv2 = wslot_v[pl.ds(c * 16, 16)]

            @pl.when(g > 0)
            def _():
                pltpu.make_async_copy(krows, kco.at[sv2], ssem).wait()
                pltpu.make_async_copy(vrows, vco.at[sv2], ssem).wait()

            gk = pltpu.async_copy(krot.at[tv], krows, gsem)
            gv = pltpu.async_copy(v.at[tv], vrows, gsem)
            gk.wait()
            gv.wait()
            pltpu.async_copy(krows, kco.at[sv2], ssem)
            pltpu.async_copy(vrows, vco.at[sv2], ssem)
        return carry

    lax.fori_loop(0, trips, trip_body, 0)

    @pl.when(trips > 0)
    def _():
        dummy = wslot_v[pl.ds(0, 16)]
        for krows, vrows, _, ssem in bufs:
            pltpu.make_async_copy(krows, kco.at[dummy], ssem).wait()
            pltpu.make_async_copy(vrows, vco.at[dummy], ssem).wait()


_scatter_call = functools.partial(
    pl.kernel,
    out_type=(),
    mesh=plsc.VectorSubcoreMesh(core_axis_name="c", subcore_axis_name="s"),
    compiler_params=pltpu.CompilerParams(
        needs_layout_passes=False,
        use_tc_tiling_on_sc=True,
    ),
    scratch_types=[
        pltpu.VMEM((LIST_LEN,), jnp.int32),
        pltpu.VMEM((LIST_LEN,), jnp.int32),
        pltpu.VMEM((16,), jnp.int32),
        pltpu.VMEM((16, N_KV, HD), jnp.float32),
        pltpu.VMEM((16, N_KV, HD), jnp.float32),
        pltpu.VMEM((16, N_KV, HD), jnp.float32),
        pltpu.VMEM((16, N_KV, HD), jnp.float32),
        pltpu.SemaphoreType.DMA,
        pltpu.SemaphoreType.DMA,
        pltpu.SemaphoreType.DMA,
        pltpu.SemaphoreType.DMA,
    ],
)(_scatter_body)


def kernel(q, k, v, positions, key_cache, value_cache, slot_mapping):
    posf = positions.astype(jnp.float32).reshape(T, 1)
    sm = slot_mapping.astype(jnp.int32)

    kco, vco = _tc_copy_call(key_cache, value_cache)
    kc_ref = jax.new_ref(kco)
    vc_ref = jax.new_ref(vco)
    wslot_h, wtok_h, cnt_h = _copy_call(key_cache, value_cache, sm,
                                        kc_ref, vc_ref)
    q_out, k_out, v_out = _rope_call(posf, q, k, v)

    _scatter_call(k_out, v_out, wslot_h, wtok_h, cnt_h, kc_ref, vc_ref)
    return (q_out, k_out, v_out, kc_ref[...], vc_ref[...])
